# Initial kernel scaffold; baseline (speedup 1.0000x reference)
import functools

import jax
import jax.numpy as jnp
from jax import lax
from jax.experimental import pallas as pl
from jax.experimental.pallas import tpu as pltpu
from jax.experimental.pallas import tpu_sc as plsc

B = 16
L = 512
D = 384
T = 2048
LP = L + 1
NW = 32
FRAMES_PER_W = (B * T) // NW
CHUNK = 128
NCH = FRAMES_PER_W // CHUNK
VL = 16


def _lr_body(xpad, dur, out, dur_v, counts_v, idx2d, buf0, buf1, gsem0, gsem1):
    cid = lax.axis_index("c")
    sid = lax.axis_index("s")
    wid = sid * 2 + cid
    b = wid // 2
    half = wid % 2

    pltpu.sync_copy(dur.at[b], dur_v)

    zeros16 = jnp.zeros((VL,), jnp.int32)
    for k in range(T // VL):
        counts_v[pl.ds(k * VL, VL)] = zeros16

    ones16 = jnp.ones((VL,), jnp.int32)
    carry = jnp.int32(0)
    for k in range(L // VL):
        dch = dur_v[pl.ds(k * VL, VL)]
        r = jnp.maximum(dch, 1)
        cs = jnp.cumsum(r) + carry
        plsc.store_scatter(counts_v, [cs], ones16, mask=cs < T)
        carry = carry + jnp.sum(r)

    base = b * LP
    acc = jnp.int32(0)
    for k in range(T // VL):
        c = counts_v[pl.ds(k * VL, VL)]
        s = jnp.cumsum(c) + (acc + base)
        idx2d[k // (CHUNK // VL), pl.ds((k % (CHUNK // VL)) * VL, VL)] = s
        acc = acc + jnp.sum(c)

    row0 = half * NCH
    obase = b * T + half * FRAMES_PER_W
    bufs = (buf0, buf1)
    gsems = (gsem0, gsem1)
    handles = [None, None]
    for ci in range(NCH):
        p = ci % 2
        handles[p] = pltpu.async_copy(xpad.at[idx2d.at[row0 + ci]], bufs[p], gsems[p])
        if ci > 0:
            q = (ci - 1) % 2
            handles[q].wait()
            pltpu.sync_copy(bufs[q], out.at[pl.ds(obase + (ci - 1) * CHUNK, CHUNK)])
    q = (NCH - 1) % 2
    handles[q].wait()
    pltpu.sync_copy(bufs[q], out.at[pl.ds(obase + (NCH - 1) * CHUNK, CHUNK)])


_lr_call = functools.partial(
    pl.kernel,
    out_type=jax.ShapeDtypeStruct((B * T, D), jnp.float32),
    mesh=plsc.VectorSubcoreMesh(core_axis_name="c", subcore_axis_name="s"),
    scratch_types=[
        pltpu.VMEM((L,), jnp.int32),
        pltpu.VMEM((T,), jnp.int32),
        pltpu.VMEM((T // CHUNK, CHUNK), jnp.int32),
        pltpu.VMEM((CHUNK, D), jnp.float32),
        pltpu.VMEM((CHUNK, D), jnp.float32),
        pltpu.SemaphoreType.DMA,
        pltpu.SemaphoreType.DMA,
    ],
)(_lr_body)


def kernel(x, durations, target_len):
    xpad = jnp.pad(x, ((0, 0), (0, 1), (0, 0))).reshape(B * LP, D)
    dur = durations.astype(jnp.int32)
    out = _lr_call(xpad, dur)
    return out.reshape(B, T, D)


# speedup vs baseline: 37.0664x; 37.0664x over previous
import functools

import jax
import jax.numpy as jnp
from jax import lax
from jax.experimental import pallas as pl
from jax.experimental.pallas import tpu as pltpu
from jax.experimental.pallas import tpu_sc as plsc

B = 16
L = 512
D = 384
T = 2048
LP = L + 1
NW = 32
FRAMES_PER_W = (B * T) // NW
CHUNK = 128
NCH = FRAMES_PER_W // CHUNK
VL = 16


def _lr_body(xpad, dur, out, dur_v, counts_v, idx2d, buf0, buf1, gsem0, gsem1):
    cid = lax.axis_index("c")
    sid = lax.axis_index("s")
    wid = sid * 2 + cid
    b = wid // 2
    half = wid % 2

    pltpu.sync_copy(dur.at[b], dur_v)

    zeros16 = jnp.zeros((VL,), jnp.int32)
    for k in range(T // VL):
        counts_v[pl.ds(k * VL, VL)] = zeros16

    ones16 = jnp.ones((VL,), jnp.int32)
    carry = jnp.int32(0)
    for k in range(L // VL):
        dch = dur_v[pl.ds(k * VL, VL)]
        r = jnp.maximum(dch, 1)
        cs = jnp.cumsum(r) + carry
        plsc.store_scatter(counts_v, [cs], ones16, mask=cs < T)
        carry = carry + jnp.sum(r)

    base = b * LP
    acc = jnp.int32(0)
    for k in range(T // VL):
        c = counts_v[pl.ds(k * VL, VL)]
        s = jnp.cumsum(c) + (acc + base)
        idx2d[k // (CHUNK // VL), pl.ds((k % (CHUNK // VL)) * VL, VL)] = s
        acc = acc + jnp.sum(c)

    row0 = half * NCH
    obase = b * T + half * FRAMES_PER_W
    bufs = (buf0, buf1)
    gsems = (gsem0, gsem1)
    handles = [None, None]
    for ci in range(NCH):
        p = ci % 2
        handles[p] = pltpu.async_copy(xpad.at[idx2d.at[row0 + ci]], bufs[p], gsems[p])
        if ci > 0:
            q = (ci - 1) % 2
            handles[q].wait()
            pltpu.sync_copy(bufs[q], out.at[pl.ds(obase + (ci - 1) * CHUNK, CHUNK)])
    q = (NCH - 1) % 2
    handles[q].wait()
    pltpu.sync_copy(bufs[q], out.at[pl.ds(obase + (NCH - 1) * CHUNK, CHUNK)])


_lr_call = functools.partial(
    pl.kernel,
    out_type=jax.ShapeDtypeStruct((B * T, D), jnp.float32),
    mesh=plsc.VectorSubcoreMesh(core_axis_name="c", subcore_axis_name="s"),
    compiler_params=pltpu.CompilerParams(needs_layout_passes=False),
    scratch_types=[
        pltpu.VMEM((L,), jnp.int32),
        pltpu.VMEM((T,), jnp.int32),
        pltpu.VMEM((T // CHUNK, CHUNK), jnp.int32),
        pltpu.VMEM((CHUNK, D), jnp.float32),
        pltpu.VMEM((CHUNK, D), jnp.float32),
        pltpu.SemaphoreType.DMA,
        pltpu.SemaphoreType.DMA,
    ],
)(_lr_body)


def kernel(x, durations, target_len):
    xpad = jnp.pad(x, ((0, 0), (0, 1), (0, 0))).reshape(B * LP, D)
    dur = durations.astype(jnp.int32)
    out = _lr_call(xpad, dur)
    return out.reshape(B, T, D)


# speedup vs baseline: 37.2356x; 1.0046x over previous
import functools

import jax
import jax.numpy as jnp
from jax import lax
from jax.experimental import pallas as pl
from jax.experimental.pallas import tpu as pltpu
from jax.experimental.pallas import tpu_sc as plsc

B = 16
L = 512
D = 384
T = 2048
LP = L + 1
NW = 32
FRAMES_PER_W = (B * T) // NW
CHUNK = 128
NCH = FRAMES_PER_W // CHUNK
VL = 16


def _lr_body(xpad, dur, out, dur_v, counts_v, idx2d, buf0, buf1,
             gsem0, gsem1, psem0, psem1):
    cid = lax.axis_index("c")
    sid = lax.axis_index("s")
    wid = sid * 2 + cid
    b = wid // 2
    half = wid % 2

    pltpu.sync_copy(dur.at[b], dur_v)

    zeros16 = jnp.zeros((VL,), jnp.int32)
    for k in range(T // VL):
        counts_v[pl.ds(k * VL, VL)] = zeros16

    ones16 = jnp.ones((VL,), jnp.int32)
    carry = jnp.int32(0)
    for k in range(L // VL):
        dch = dur_v[pl.ds(k * VL, VL)]
        r = jnp.maximum(dch, 1)
        cs = jnp.cumsum(r) + carry
        plsc.store_scatter(counts_v, [cs], ones16, mask=cs < T)
        carry = carry + jnp.sum(r)

    base = b * LP
    acc = jnp.int32(0)
    for k in range(T // VL):
        c = counts_v[pl.ds(k * VL, VL)]
        s = jnp.cumsum(c) + (acc + base)
        idx2d[k // (CHUNK // VL), pl.ds((k % (CHUNK // VL)) * VL, VL)] = s
        acc = acc + jnp.sum(c)

    row0 = half * NCH
    obase = b * T + half * FRAMES_PER_W
    bufs = (buf0, buf1)
    gsems = (gsem0, gsem1)
    psems = (psem0, psem1)
    ghandles = [None, None]
    phandles = [None, None]
    for ci in range(NCH):
        p = ci % 2
        if ci >= 2:
            phandles[p].wait()
        ghandles[p] = pltpu.async_copy(xpad.at[idx2d.at[row0 + ci]], bufs[p], gsems[p])
        if ci > 0:
            q = (ci - 1) % 2
            ghandles[q].wait()
            phandles[q] = pltpu.async_copy(
                bufs[q], out.at[pl.ds(obase + (ci - 1) * CHUNK, CHUNK)], psems[q])
    q = (NCH - 1) % 2
    ghandles[q].wait()
    phandles[q] = pltpu.async_copy(
        bufs[q], out.at[pl.ds(obase + (NCH - 1) * CHUNK, CHUNK)], psems[q])
    phandles[0].wait()
    phandles[1].wait()


_lr_call = functools.partial(
    pl.kernel,
    out_type=jax.ShapeDtypeStruct((B * T, D), jnp.float32),
    mesh=plsc.VectorSubcoreMesh(core_axis_name="c", subcore_axis_name="s"),
    compiler_params=pltpu.CompilerParams(needs_layout_passes=False),
    scratch_types=[
        pltpu.VMEM((L,), jnp.int32),
        pltpu.VMEM((T,), jnp.int32),
        pltpu.VMEM((T // CHUNK, CHUNK), jnp.int32),
        pltpu.VMEM((CHUNK, D), jnp.float32),
        pltpu.VMEM((CHUNK, D), jnp.float32),
        pltpu.SemaphoreType.DMA,
        pltpu.SemaphoreType.DMA,
        pltpu.SemaphoreType.DMA,
        pltpu.SemaphoreType.DMA,
    ],
)(_lr_body)


def kernel(x, durations, target_len):
    xpad = jnp.pad(x, ((0, 0), (0, 1), (0, 0))).reshape(B * LP, D)
    dur = durations.astype(jnp.int32)
    out = _lr_call(xpad, dur)
    return out.reshape(B, T, D)


# speedup vs baseline: 51.1054x; 1.3725x over previous
import functools

import jax
import jax.numpy as jnp
from jax import lax
from jax.experimental import pallas as pl
from jax.experimental.pallas import tpu as pltpu
from jax.experimental.pallas import tpu_sc as plsc

B = 16
L = 512
D = 384
T = 2048
NW = 32
FRAMES_PER_W = (B * T) // NW
CHUNK = 64
NCH = FRAMES_PER_W // CHUNK
VL = 16


def _lr_body(xflat, dur, zrows, out, dur_v, counts_v, idx2d, buf0, buf1, zbuf,
             gsem0, gsem1, psem0, psem1):
    cid = lax.axis_index("c")
    sid = lax.axis_index("s")
    wid = sid * 2 + cid
    b = wid // 2
    half = wid % 2

    pltpu.sync_copy(zrows, zbuf)
    pltpu.sync_copy(dur.at[b], dur_v)

    zeros16 = jnp.zeros((VL,), jnp.int32)
    for k in range(T // VL):
        counts_v[pl.ds(k * VL, VL)] = zeros16

    ones16 = jnp.ones((VL,), jnp.int32)
    carry = jnp.int32(0)
    for k in range(L // VL):
        dch = dur_v[pl.ds(k * VL, VL)]
        r = jnp.maximum(dch, 1)
        cs = jnp.cumsum(r) + carry
        plsc.store_scatter(counts_v, [cs], ones16, mask=cs < T)
        carry = carry + jnp.sum(r)
    total = carry

    base = b * L
    acc = jnp.int32(0)
    for k in range(T // VL):
        c = counts_v[pl.ds(k * VL, VL)]
        s = jnp.minimum(jnp.cumsum(c) + (acc + base), base + L - 1)
        idx2d[k // (CHUNK // VL), pl.ds((k % (CHUNK // VL)) * VL, VL)] = s
        acc = acc + jnp.sum(c)

    row0 = half * NCH
    frame0 = half * FRAMES_PER_W
    obase = b * T + frame0
    bufs = (buf0, buf1)
    gsems = (gsem0, gsem1)
    psems = (psem0, psem1)

    def gcopy(ci, p):
        return pltpu.make_async_copy(xflat.at[idx2d.at[row0 + ci]], bufs[p], gsems[p])

    def pvalid(ci, p):
        return pltpu.make_async_copy(
            bufs[p], out.at[pl.ds(obase + ci * CHUNK, CHUNK)], psems[p])

    def pzero(ci, p):
        return pltpu.make_async_copy(
            zbuf, out.at[pl.ds(obase + ci * CHUNK, CHUNK)], psems[p])

    def start_chunk(ci, p):
        v = total - (frame0 + ci * CHUNK)
        pl.when(v > 0)(lambda: gcopy(ci, p).start())

    def finish_chunk(ci, p):
        v = total - (frame0 + ci * CHUNK)

        def valid_case():
            gcopy(ci, p).wait()

            def zero_tail():
                def zero_row(rr, _):
                    for j in range(D // VL):
                        bufs[p][rr, pl.ds(j * VL, VL)] = jnp.zeros((VL,), jnp.float32)
                    return 0
                lax.fori_loop(v, CHUNK, zero_row, 0)
            pl.when(v < CHUNK)(zero_tail)
            pvalid(ci, p).start()

        def masked_case():
            pzero(ci, p).start()

        pl.when(v > 0)(valid_case)
        pl.when(v <= 0)(masked_case)

    for ci in range(NCH):
        p = ci % 2
        if ci >= 2:
            pvalid(ci - 2, p).wait()
        start_chunk(ci, p)
        if ci > 0:
            finish_chunk(ci - 1, (ci - 1) % 2)
    finish_chunk(NCH - 1, (NCH - 1) % 2)
    pvalid(NCH - 2, (NCH - 2) % 2).wait()
    pvalid(NCH - 1, (NCH - 1) % 2).wait()


_lr_call = functools.partial(
    pl.kernel,
    out_type=jax.ShapeDtypeStruct((B * T, D), jnp.float32),
    mesh=plsc.VectorSubcoreMesh(core_axis_name="c", subcore_axis_name="s"),
    compiler_params=pltpu.CompilerParams(needs_layout_passes=False),
    scratch_types=[
        pltpu.VMEM((L,), jnp.int32),
        pltpu.VMEM((T,), jnp.int32),
        pltpu.VMEM((T // CHUNK, CHUNK), jnp.int32),
        pltpu.VMEM((CHUNK, D), jnp.float32),
        pltpu.VMEM((CHUNK, D), jnp.float32),
        pltpu.VMEM((CHUNK, D), jnp.float32),
        pltpu.SemaphoreType.DMA,
        pltpu.SemaphoreType.DMA,
        pltpu.SemaphoreType.DMA,
        pltpu.SemaphoreType.DMA,
    ],
)(_lr_body)


def kernel(x, durations, target_len):
    xflat = x.reshape(B * L, D)
    dur = durations.astype(jnp.int32)
    zrows = jnp.zeros((CHUNK, D), jnp.float32)
    out = _lr_call(xflat, dur, zrows)
    return out.reshape(B, T, D)
